# Initial kernel scaffold; baseline (speedup 1.0000x reference)
import jax
import jax.numpy as jnp
from jax.experimental import pallas as pl


def kernel(predicted, target):
    raise NotImplementedError("write your pallas kernel here")


# speedup vs baseline: 1.0065x; 1.0065x over previous
import functools

import jax
import jax.numpy as jnp
from jax.experimental import pallas as pl
from jax.experimental.pallas import tpu as pltpu

_NCORES = 2
_TILE_BYTES = 4 * 1024 * 1024


def _sse_partial_kernel(p_ref, t_ref, out_ref, acc_ref, *, scale, tile_rows,
                        rows, exact):
    j = pl.program_id(1)
    nj = pl.num_programs(1)

    @pl.when(j == 0)
    def _():
        acc_ref[...] = jnp.zeros_like(acc_ref)

    d = p_ref[...] - t_ref[...]
    sq = d * d

    if not exact:
        i = pl.program_id(0)
        start = (i * nj + j) * tile_rows
        limit = rows - start
        row_ids = jax.lax.broadcasted_iota(jnp.int32, sq.shape, 0)
        sq = jnp.where(row_ids < limit, sq, 0.0)

    acc_ref[...] += jnp.sum(sq, axis=0, keepdims=True)

    @pl.when(j == nj - 1)
    def _():
        total = jnp.sum(acc_ref[...]) * jnp.float32(scale)
        out_ref[...] = total.reshape(1, 1, 1)


@jax.jit
def kernel(predicted, target):
    assert predicted.shape == target.shape
    n_elems = predicted.size
    scale = 10000.0 / float(n_elems)

    lane = next((c for c in (512, 256, 128) if n_elems % c == 0), None)
    if lane is None:
        d = predicted.astype(jnp.float32) - target.astype(jnp.float32)
        return jnp.mean(d * d) * jnp.float32(10000.0)

    p2 = predicted.reshape(-1, lane).astype(jnp.float32)
    t2 = target.reshape(-1, lane).astype(jnp.float32)
    rows = p2.shape[0]

    tile_rows = max(8, min(rows, _TILE_BYTES // (4 * lane)) // 8 * 8)
    num_tiles = -(-rows // tile_rows)
    nj = -(-num_tiles // _NCORES)
    exact = (rows == tile_rows * nj * _NCORES)
    last_tile = num_tiles - 1

    partials = pl.pallas_call(
        functools.partial(
            _sse_partial_kernel,
            scale=scale,
            tile_rows=tile_rows,
            rows=rows,
            exact=exact,
        ),
        out_shape=jax.ShapeDtypeStruct((_NCORES, 1, 1), jnp.float32),
        grid=(_NCORES, nj),
        in_specs=[
            pl.BlockSpec((tile_rows, lane),
                         lambda i, j: (jnp.minimum(i * nj + j, last_tile), 0)),
            pl.BlockSpec((tile_rows, lane),
                         lambda i, j: (jnp.minimum(i * nj + j, last_tile), 0)),
        ],
        out_specs=pl.BlockSpec((1, 1, 1), lambda i, j: (i, 0, 0)),
        scratch_shapes=[pltpu.VMEM((1, lane), jnp.float32)],
        compiler_params=pltpu.CompilerParams(
            dimension_semantics=("parallel", "arbitrary"),
        ),
    )(p2, t2)

    return partials[0, 0, 0] + partials[1, 0, 0]


# speedup vs baseline: 1.0175x; 1.0109x over previous
import functools

import jax
import jax.numpy as jnp
from jax.experimental import pallas as pl
from jax.experimental.pallas import tpu as pltpu

_NSLICES = 4
_SLICE_BYTES = 2 * 1024 * 1024


def _sse_kernel(*refs, scale, tile_rows, rows, nsl, exact):
    p_refs = refs[:nsl]
    t_refs = refs[nsl:2 * nsl]
    out_ref = refs[2 * nsl]
    acc_ref = refs[2 * nsl + 1]
    j = pl.program_id(0)
    nj = pl.num_programs(0)

    @pl.when(j == 0)
    def _():
        acc_ref[...] = jnp.zeros_like(acc_ref)

    partial = None
    for s in range(nsl):
        d = p_refs[s][...] - t_refs[s][...]
        sq = d * d
        if not exact:
            limit = rows - (j * nsl + s) * tile_rows
            row_ids = jax.lax.broadcasted_iota(jnp.int32, sq.shape, 0)
            sq = jnp.where(row_ids < limit, sq, 0.0)
        ps = jnp.sum(sq, axis=0, keepdims=True)
        partial = ps if partial is None else partial + ps
    acc_ref[...] += partial

    @pl.when(j == nj - 1)
    def _():
        out_ref[...] = jnp.sum(acc_ref[...], keepdims=True) * jnp.float32(scale)


@jax.jit
def kernel(predicted, target):
    assert predicted.shape == target.shape
    n_elems = predicted.size
    scale = 10000.0 / float(n_elems)

    lane = next((c for c in (512, 256, 128) if n_elems % c == 0), None)
    if lane is None:
        d = predicted.astype(jnp.float32) - target.astype(jnp.float32)
        return jnp.mean(d * d) * jnp.float32(10000.0)

    p2 = predicted.reshape(-1, lane).astype(jnp.float32)
    t2 = target.reshape(-1, lane).astype(jnp.float32)
    rows = p2.shape[0]

    tile_rows = max(8, min(rows, _SLICE_BYTES // (4 * lane)) // 8 * 8)
    num_tiles = -(-rows // tile_rows)
    nj = -(-num_tiles // _NSLICES)
    exact = (rows == tile_rows * nj * _NSLICES)
    last_tile = num_tiles - 1

    def _slice_spec(s):
        return pl.BlockSpec(
            (tile_rows, lane),
            lambda j, s=s: (jnp.minimum(j * _NSLICES + s, last_tile), 0),
        )

    loss = pl.pallas_call(
        functools.partial(
            _sse_kernel,
            scale=scale,
            tile_rows=tile_rows,
            rows=rows,
            nsl=_NSLICES,
            exact=exact,
        ),
        out_shape=jax.ShapeDtypeStruct((1, 1), jnp.float32),
        grid=(nj,),
        in_specs=[_slice_spec(s) for s in range(_NSLICES)] * 2,
        out_specs=pl.BlockSpec((1, 1), lambda j: (0, 0)),
        scratch_shapes=[pltpu.VMEM((1, lane), jnp.float32)],
        compiler_params=pltpu.CompilerParams(
            dimension_semantics=("arbitrary",),
        ),
    )(p2, p2, p2, p2, t2, t2, t2, t2)

    return loss[0, 0]
